# baseline (speedup 1.0000x reference)
import functools

import jax
import jax.numpy as jnp
from jax import lax
from jax.experimental import pallas as pl
from jax.experimental.pallas import tpu as pltpu
from jax.experimental.pallas import tpu_sc as plsc

_NC = 2
_NS = 16
_NW = _NC * _NS
_LANES = 16
_UNROLL = 8

_SC_FRAC_NUM = 4
_TC_BLOCK = 1048576
_SC_CHUNK = 8192


def _sc_tail(n, tail, chunk):
    base0 = n - tail
    per_w = tail // _NW
    n_chunks = per_w // chunk
    n_pairs = n_chunks // 2
    mesh = plsc.VectorSubcoreMesh(core_axis_name="c", subcore_axis_name="s")

    @functools.partial(
        pl.kernel,
        mesh=mesh,
        out_type=jax.ShapeDtypeStruct((tail,), jnp.float32),
        scratch_types=[
            pltpu.VMEM((chunk,), jnp.float32),
            pltpu.VMEM((chunk,), jnp.float32),
            pltpu.VMEM((chunk,), jnp.float32),
            pltpu.VMEM((chunk,), jnp.float32),
            pltpu.VMEM((chunk,), jnp.float32),
            pltpu.VMEM((chunk,), jnp.float32),
            pltpu.SemaphoreType.DMA,
            pltpu.SemaphoreType.DMA,
            pltpu.SemaphoreType.DMA,
            pltpu.SemaphoreType.DMA,
            pltpu.SemaphoreType.DMA,
            pltpu.SemaphoreType.DMA,
        ],
    )
    def k(x_hbm, y_hbm, o_hbm,
          xv0, xv1, yv0, yv1, ov0, ov1,
          sx0, sx1, sy0, sy1, so0, so1):
        wid = lax.axis_index("s") * _NC + lax.axis_index("c")
        obase = wid * per_w
        ibase = base0 + obase
        xvs, yvs, ovs = (xv0, xv1), (yv0, yv1), (ov0, ov1)
        sxs, sys_, sos = (sx0, sx1), (sy0, sy1), (so0, so1)

        def load(i, s):
            off = ibase + i * chunk
            pltpu.make_async_copy(
                x_hbm.at[pl.ds(off, chunk)], xvs[s], sxs[s]).start()
            pltpu.make_async_copy(
                y_hbm.at[pl.ds(off, chunk)], yvs[s], sys_[s]).start()

        def wait_load(s):
            pltpu.make_async_copy(
                x_hbm.at[pl.ds(0, chunk)], xvs[s], sxs[s]).wait()
            pltpu.make_async_copy(
                y_hbm.at[pl.ds(0, chunk)], yvs[s], sys_[s]).wait()

        def store(i, s):
            off = obase + i * chunk
            pltpu.make_async_copy(
                ovs[s], o_hbm.at[pl.ds(off, chunk)], sos[s]).start()

        def wait_store(s):
            pltpu.make_async_copy(
                ovs[s], o_hbm.at[pl.ds(0, chunk)], sos[s]).wait()

        def compute(s):
            xv, yv, ov = xvs[s], yvs[s], ovs[s]

            def body(j, c):
                for u in range(_UNROLL):
                    sl = pl.ds((j * _UNROLL + u) * _LANES, _LANES)
                    yy = yv[sl]
                    inv = 1.0 / jnp.where(yy == 0.0, jnp.inf, yy)
                    ov[sl] = inv * xv[sl]
                return c

            lax.fori_loop(0, chunk // (_LANES * _UNROLL), body, 0)

        load(0, 0)
        load(1, 1)

        def pair_body(t, c):
            for s in range(2):
                i = 2 * t + s
                wait_load(s)
                pl.when(t > 0)(lambda s=s: wait_store(s))
                compute(s)
                store(i, s)
                pl.when(t < n_pairs - 1)(lambda i=i, s=s: load(i + 2, s))
            return c

        lax.fori_loop(0, n_pairs, pair_body, 0)
        wait_store(0)
        wait_store(1)

    return k


def _tc_body(x_ref, y_ref, o_ref):
    y = y_ref[...]
    inv = 1.0 / jnp.where(y == 0.0, jnp.inf, y)
    o_ref[...] = inv * x_ref[...]


def kernel(x, y):
    n = x.shape[0]
    return _sc_tail(n, n, _SC_CHUNK)(x, y)


# speedup vs baseline: 1.0463x; 1.0463x over previous
import functools

import jax
import jax.numpy as jnp
from jax import lax
from jax.experimental import pallas as pl
from jax.experimental.pallas import tpu as pltpu
from jax.experimental.pallas import tpu_sc as plsc

_NC = 2
_NS = 16
_NW = _NC * _NS
_LANES = 16
_UNROLL = 8
_CHUNK = 16384


def _sc_stable_zero_div(n, chunk):
    per_w = n // _NW
    n_chunks = per_w // chunk
    n_pairs = n_chunks // 2
    mesh = plsc.VectorSubcoreMesh(core_axis_name="c", subcore_axis_name="s")

    @functools.partial(
        pl.kernel,
        mesh=mesh,
        out_type=jax.ShapeDtypeStruct((n,), jnp.float32),
        scratch_types=[
            pltpu.VMEM((chunk,), jnp.float32),
            pltpu.VMEM((chunk,), jnp.float32),
            pltpu.VMEM((chunk,), jnp.float32),
            pltpu.VMEM((chunk,), jnp.float32),
            pltpu.VMEM((chunk,), jnp.float32),
            pltpu.VMEM((chunk,), jnp.float32),
            pltpu.SemaphoreType.DMA,
            pltpu.SemaphoreType.DMA,
            pltpu.SemaphoreType.DMA,
            pltpu.SemaphoreType.DMA,
            pltpu.SemaphoreType.DMA,
            pltpu.SemaphoreType.DMA,
        ],
    )
    def k(x_hbm, y_hbm, o_hbm,
          xv0, xv1, yv0, yv1, ov0, ov1,
          sx0, sx1, sy0, sy1, so0, so1):
        wid = lax.axis_index("s") * _NC + lax.axis_index("c")
        base = wid * per_w
        xvs, yvs, ovs = (xv0, xv1), (yv0, yv1), (ov0, ov1)
        sxs, sys_, sos = (sx0, sx1), (sy0, sy1), (so0, so1)

        def load(i, s):
            off = base + i * chunk
            pltpu.make_async_copy(
                x_hbm.at[pl.ds(off, chunk)], xvs[s], sxs[s]).start()
            pltpu.make_async_copy(
                y_hbm.at[pl.ds(off, chunk)], yvs[s], sys_[s]).start()

        def wait_load(s):
            pltpu.make_async_copy(
                x_hbm.at[pl.ds(0, chunk)], xvs[s], sxs[s]).wait()
            pltpu.make_async_copy(
                y_hbm.at[pl.ds(0, chunk)], yvs[s], sys_[s]).wait()

        def store(i, s):
            off = base + i * chunk
            pltpu.make_async_copy(
                ovs[s], o_hbm.at[pl.ds(off, chunk)], sos[s]).start()

        def wait_store(s):
            pltpu.make_async_copy(
                ovs[s], o_hbm.at[pl.ds(0, chunk)], sos[s]).wait()

        def compute(s):
            xv, yv, ov = xvs[s], yvs[s], ovs[s]

            def body(j, c):
                for u in range(_UNROLL):
                    sl = pl.ds((j * _UNROLL + u) * _LANES, _LANES)
                    yy = yv[sl]
                    inv = 1.0 / jnp.where(yy == 0.0, jnp.inf, yy)
                    ov[sl] = inv * xv[sl]
                return c

            lax.fori_loop(0, chunk // (_LANES * _UNROLL), body, 0)

        load(0, 0)
        load(1, 1)

        def pair_body(t, c):
            for s in range(2):
                i = 2 * t + s
                wait_load(s)
                pl.when(t > 0)(lambda s=s: wait_store(s))
                compute(s)
                store(i, s)
                pl.when(t < n_pairs - 1)(lambda i=i, s=s: load(i + 2, s))
            return c

        lax.fori_loop(0, n_pairs, pair_body, 0)
        wait_store(0)
        wait_store(1)

    return k


def kernel(x, y):
    n = x.shape[0]
    return _sc_stable_zero_div(n, _CHUNK)(x, y)
